# baseline (speedup 1.0000x reference)
import functools

import jax
import jax.numpy as jnp
from jax import lax
from jax.experimental import pallas as pl
from jax.experimental.pallas import tpu as pltpu
from jax.experimental.pallas import tpu_sc as plsc

N = 10000
NP = 10240
E = 320000
D = 128
H = 16

NC = 2
NS = 16
NW = NC * NS
RPT = NP // NS
CHUNK = 128
EROWS = E // CHUNK
NROW = EROWS // NW
XROW = EROWS - NROW * NW
RING = 13
LOOK = 8

HALF = NP // NC
NR2 = EROWS // NS
XR2 = EROWS - NR2 * NS
OPT = HALF // NS

_MESH = plsc.VectorSubcoreMesh(
    core_axis_name="c", subcore_axis_name="s", num_cores=NC, num_subcores=NS
)
_SC_PARAMS = pltpu.CompilerParams(use_tc_tiling_on_sc=False,
                                  needs_layout_passes=False)


def _zero_rows(ref, nrows, width):
    if width == 1:
        def body(i, carry):
            ref[pl.ds(i * 16, 16)] = jnp.zeros((16,), jnp.float32)
            return carry
        lax.fori_loop(0, nrows // 16, body, 0)
    else:
        def body(i, carry):
            for j in range(width // 16):
                ref[i, pl.ds(j * 16, 16)] = jnp.zeros((16,), jnp.float32)
            return carry
        lax.fori_loop(0, nrows, body, 0)


def _make_edge_agg(mode):
    width = H if mode == "stream" else 1
    if width == 1:
        out_t = jax.ShapeDtypeStruct((NC, NP), jnp.float32)
        rows_t = pltpu.VMEM((RING, CHUNK), jnp.float32)
        zb_t = pltpu.VMEM((RPT,), jnp.float32)
        acc_t = pltpu.VMEM_SHARED((NP,), jnp.float32)
    else:
        out_t = jax.ShapeDtypeStruct((NC, NP, width), jnp.float32)
        rows_t = pltpu.VMEM((RING, CHUNK, width), jnp.float32)
        zb_t = pltpu.VMEM((RPT, width), jnp.float32)
        acc_t = pltpu.VMEM_SHARED((NP, width), jnp.float32)

    scratch = [pltpu.VMEM((NROW + 1, CHUNK), jnp.int32)]
    if mode != "deg":
        scratch.append(pltpu.VMEM((NROW + 1, CHUNK), jnp.int32))
    if mode == "vreg":
        scratch.append(pltpu.VMEM((N,), jnp.float32))
    scratch += [rows_t, zb_t, acc_t]
    scratch += [pltpu.SemaphoreType.DMA for _ in range(RING)]
    if mode == "stream":
        scratch += [pltpu.SemaphoreType.DMA for _ in range(RING)]

    @functools.partial(
        pl.kernel,
        out_type=out_t,
        mesh=_MESH,
        compiler_params=_SC_PARAMS,
        scratch_types=scratch,
    )
    def agg(*refs):
        it = iter(refs)
        if mode == "deg":
            dst2_hbm, out_hbm = next(it), next(it)
        else:
            table_hbm, src2_hbm, dst2_hbm, out_hbm = (
                next(it), next(it), next(it), next(it))
        idx_d2 = next(it)
        if mode != "deg":
            idx_s2 = next(it)
        if mode == "vreg":
            tab_v = next(it)
        rows_v, zb_v, acc_sh = next(it), next(it), next(it)
        ssem = [next(it) for _ in range(RING)]
        if mode == "stream":
            gsem = [next(it) for _ in range(RING)]

        cid = lax.axis_index("c")
        sid = lax.axis_index("s")
        wid = cid * NS + sid
        has_extra = wid < XROW

        pltpu.sync_copy(dst2_hbm.at[pl.ds(wid * NROW, NROW), :],
                        idx_d2.at[pl.ds(0, NROW), :])
        if mode != "deg":
            pltpu.sync_copy(src2_hbm.at[pl.ds(wid * NROW, NROW), :],
                            idx_s2.at[pl.ds(0, NROW), :])

        @pl.when(has_extra)
        def _():
            xr = NW * NROW + wid
            pltpu.sync_copy(dst2_hbm.at[xr], idx_d2.at[NROW])
            if mode != "deg":
                pltpu.sync_copy(src2_hbm.at[xr], idx_s2.at[NROW])

        if mode == "deg":
            for j in range(CHUNK // 16):
                rows_v[0, pl.ds(j * 16, 16)] = jnp.ones((16,), jnp.float32)
        if mode == "vreg":
            pltpu.sync_copy(table_hbm, tab_v)

        _zero_rows(zb_v, RPT, width)
        pltpu.sync_copy(zb_v, acc_sh.at[pl.ds(sid * RPT, RPT)])
        plsc.subcore_barrier()

        def gather_start(c, slot):
            pltpu.async_copy(table_hbm.at[idx_s2.at[c]], rows_v.at[slot],
                             gsem[slot])

        def gather_wait(c, slot):
            pltpu.make_async_copy(table_hbm.at[idx_s2.at[c]],
                                  rows_v.at[slot], gsem[slot]).wait()

        def vreg_fill(c, slot):
            for k in range(CHUNK // 16):
                sv = idx_s2[c, pl.ds(k * 16, 16)]
                rows_v[slot, pl.ds(k * 16, 16)] = plsc.load_gather(
                    tab_v, [sv])

        def scatter_start(c, slot):
            src = rows_v.at[slot] if mode != "deg" else rows_v.at[0]
            pltpu.async_copy(src, acc_sh.at[idx_d2.at[c]], ssem[slot],
                             add=True)

        def scatter_wait(c, slot):
            src = rows_v.at[slot] if mode != "deg" else rows_v.at[0]
            pltpu.make_async_copy(src, acc_sh.at[idx_d2.at[c]],
                                  ssem[slot]).wait()

        if mode == "stream":
            for j in range(LOOK):
                gather_start(j, j)

            def outer(g, carry):
                for j in range(RING):
                    i = g * RING + j
                    look_slot = (j + LOOK) % RING
                    c = i + LOOK

                    @pl.when(c < NROW)
                    def _():
                        @pl.when(c >= RING)
                        def _():
                            scatter_wait(c, look_slot)
                        gather_start(c, look_slot)

                    gather_wait(i, j)
                    scatter_start(i, j)
                return carry

            lax.fori_loop(0, NROW // RING, outer, 0)
        else:
            def outer(g, carry):
                for j in range(RING):
                    i = g * RING + j

                    @pl.when(i >= RING)
                    def _():
                        scatter_wait(i, j)
                    if mode == "vreg":
                        vreg_fill(i, j)
                    scatter_start(i, j)
                return carry

            lax.fori_loop(0, NROW // RING, outer, 0)

        for j in range(RING):
            scatter_wait(0, j)

        @pl.when(has_extra)
        def _():
            if mode == "stream":
                gather_start(NROW, 0)
                gather_wait(NROW, 0)
            if mode == "vreg":
                vreg_fill(NROW, 0)
            scatter_start(NROW, 0)
            scatter_wait(NROW, 0)

        plsc.subcore_barrier()
        pltpu.sync_copy(
            acc_sh.at[pl.ds(sid * RPT, RPT)],
            out_hbm.at[cid, pl.ds(sid * RPT, RPT)],
        )

    return agg


_deg_kernel = _make_edge_agg("deg")
_agg16 = _make_edge_agg("stream")


@functools.partial(
    pl.kernel,
    out_type=jax.ShapeDtypeStruct((N,), jnp.float32),
    mesh=_MESH,
    compiler_params=_SC_PARAMS,
    scratch_types=[
        pltpu.VMEM((NR2 + 1, CHUNK), jnp.int32),
        pltpu.VMEM((NR2 + 1, CHUNK), jnp.int32),
        pltpu.VMEM((NP,), jnp.float32),
        pltpu.VMEM((OPT,), jnp.float32),
        pltpu.VMEM((16,), jnp.float32),
        pltpu.VMEM((RING, CHUNK), jnp.float32),
        pltpu.VMEM((RPT,), jnp.float32),
        pltpu.VMEM((OPT,), jnp.float32),
        pltpu.VMEM_SHARED((NP,), jnp.float32),
    ] + [pltpu.SemaphoreType.DMA for _ in range(RING)],
)
def _agg1_final(t2_hbm, dis_hbm, b2_hbm, src2_hbm, dst2_hbm, out_hbm,
                idx_d2, idx_s2, tab_v, dis_v, b2_v, rows_v, zb_v, out_v,
                acc_sh, *ssem):
    cid = lax.axis_index("c")
    sid = lax.axis_index("s")
    lo = cid * HALF
    has_extra = sid < XR2

    pltpu.sync_copy(dst2_hbm.at[pl.ds(sid * NR2, NR2), :],
                    idx_d2.at[pl.ds(0, NR2), :])
    pltpu.sync_copy(src2_hbm.at[pl.ds(sid * NR2, NR2), :],
                    idx_s2.at[pl.ds(0, NR2), :])

    @pl.when(has_extra)
    def _():
        xr = NS * NR2 + sid
        pltpu.sync_copy(dst2_hbm.at[xr], idx_d2.at[NR2])
        pltpu.sync_copy(src2_hbm.at[xr], idx_s2.at[NR2])

    pltpu.sync_copy(t2_hbm, tab_v)
    pltpu.sync_copy(dis_hbm.at[pl.ds(lo + sid * OPT, OPT)], dis_v)
    pltpu.sync_copy(b2_hbm, b2_v)

    _zero_rows(zb_v, RPT, 1)
    pltpu.sync_copy(zb_v, acc_sh.at[pl.ds(sid * RPT, RPT)])
    plsc.subcore_barrier()

    def fill(c, slot):
        for k in range(CHUNK // 16):
            sv = idx_s2[c, pl.ds(k * 16, 16)]
            rows_v[slot, pl.ds(k * 16, 16)] = plsc.load_gather(tab_v, [sv])

    def scatter_start(c, slot):
        pltpu.async_copy(rows_v.at[slot], acc_sh.at[idx_d2.at[c]],
                         ssem[slot], add=True)

    def scatter_wait(c, slot):
        pltpu.make_async_copy(rows_v.at[slot], acc_sh.at[idx_d2.at[c]],
                              ssem[slot]).wait()

    def outer(g, carry):
        for j in range(RING):
            i = g * RING + j

            @pl.when(i >= RING)
            def _():
                scatter_wait(i, j)
            fill(i, j)
            scatter_start(i, j)
        return carry

    lax.fori_loop(0, NR2 // RING, outer, 0)
    for j in range(RING):
        scatter_wait(0, j)

    @pl.when(has_extra)
    def _():
        fill(NR2, 0)
        scatter_start(NR2, 0)
        scatter_wait(NR2, 0)

    plsc.subcore_barrier()

    pltpu.sync_copy(acc_sh.at[pl.ds(lo + sid * OPT, OPT)], out_v)
    b2vec = b2_v[...]
    nb = lo + sid * OPT
    for k in range(OPT // 16):
        sl = pl.ds(k * 16, 16)
        t2l = tab_v[pl.ds(nb + k * 16, 16)]
        out_v[sl] = dis_v[sl] * (out_v[sl] + t2l) + b2vec

    base = lo + sid * OPT

    @pl.when(base + OPT <= N)
    def _():
        pltpu.sync_copy(out_v, out_hbm.at[pl.ds(base, OPT)])

    @pl.when(base + OPT > N)
    def _():
        pltpu.sync_copy(out_v.at[pl.ds(0, N - (NC * NS - 1) * OPT)],
                        out_hbm.at[pl.ds(base, N - (NC * NS - 1) * OPT)])


def _tc_matmul(x, w1):
    def body(x_ref, w_ref, h_ref):
        h_ref[...] = jnp.dot(x_ref[...], w_ref[...],
                             preferred_element_type=jnp.float32)

    return pl.pallas_call(
        body,
        out_shape=jax.ShapeDtypeStruct((N, H), jnp.float32),
    )(x, w1)


def _tc_stage1(h, d0, d1):
    def body(h_ref, d0_ref, d1_ref, dis_ref, g1_ref):
        dis = lax.rsqrt(d0_ref[0:N, :] + d1_ref[0:N, :] + 1.0)
        dis_ref[0:N, :] = dis
        g1_ref[...] = dis * h_ref[...]

    return pl.pallas_call(
        body,
        out_shape=[
            jax.ShapeDtypeStruct((NP, 1), jnp.float32),
            jax.ShapeDtypeStruct((N, H), jnp.float32),
        ],
    )(h, d0, d1)


def _tc_stage2(acc0, acc1, g1, dis, b1, w2):
    def body(a0_ref, a1_ref, g1_ref, dis_ref, b1_ref, w2_ref, t2_ref):
        agg = a0_ref[0:N, :] + a1_ref[0:N, :] + g1_ref[...]
        disn = dis_ref[0:N, :]
        out1 = disn * agg + b1_ref[...]
        h1 = jnp.maximum(out1, 0.0)
        g2 = jnp.dot(h1, w2_ref[...], preferred_element_type=jnp.float32)
        t2_ref[0:N, :] = disn * g2

    return pl.pallas_call(
        body,
        out_shape=jax.ShapeDtypeStruct((NP, 1), jnp.float32),
    )(acc0, acc1, g1, dis, b1, w2)


def kernel(x, edge_index, W1, b1, W2, b2):
    src2 = edge_index[0].reshape(EROWS, CHUNK)
    dst2 = edge_index[1].reshape(EROWS, CHUNK)

    h = _tc_matmul(x, W1)
    degp = _deg_kernel(dst2)
    dis, g1 = _tc_stage1(h,
                         degp[0].reshape(NP, 1),
                         degp[1].reshape(NP, 1))
    accp = _agg16(g1, src2, dst2)
    t2 = _tc_stage2(accp[0], accp[1], g1, dis,
                    b1.reshape(1, H), W2)
    out = _agg1_final(t2.reshape(NP), dis.reshape(NP),
                      jnp.tile(b2, 16), src2, dst2)
    return out.reshape(N, 1)


# speedup vs baseline: 1.0066x; 1.0066x over previous
import functools

import jax
import jax.numpy as jnp
from jax import lax
from jax.experimental import pallas as pl
from jax.experimental.pallas import tpu as pltpu
from jax.experimental.pallas import tpu_sc as plsc

N = 10000
NP = 10240
E = 320000
D = 128
H = 16

NC = 2
NS = 16
NW = NC * NS
RPT = NP // NS
CHUNK = 128
EROWS = E // CHUNK
NROW = EROWS // NW
XROW = EROWS - NROW * NW
RING = 13
LOOK = 10

HALF = NP // NC
NR2 = EROWS // NS
XR2 = EROWS - NR2 * NS
OPT = HALF // NS

_MESH = plsc.VectorSubcoreMesh(
    core_axis_name="c", subcore_axis_name="s", num_cores=NC, num_subcores=NS
)
_SC_PARAMS = pltpu.CompilerParams(use_tc_tiling_on_sc=False,
                                  needs_layout_passes=False)


def _zero_rows(ref, nrows, width):
    if width == 1:
        def body(i, carry):
            ref[pl.ds(i * 16, 16)] = jnp.zeros((16,), jnp.float32)
            return carry
        lax.fori_loop(0, nrows // 16, body, 0)
    else:
        def body(i, carry):
            for j in range(width // 16):
                ref[i, pl.ds(j * 16, 16)] = jnp.zeros((16,), jnp.float32)
            return carry
        lax.fori_loop(0, nrows, body, 0)


def _make_edge_agg(mode):
    width = H if mode == "stream" else 1
    if width == 1:
        out_t = jax.ShapeDtypeStruct((NC, NP), jnp.float32)
        rows_t = pltpu.VMEM((RING, CHUNK), jnp.float32)
        zb_t = pltpu.VMEM((RPT,), jnp.float32)
        acc_t = pltpu.VMEM_SHARED((NP,), jnp.float32)
    else:
        out_t = jax.ShapeDtypeStruct((NC, NP, width), jnp.float32)
        rows_t = pltpu.VMEM((RING, CHUNK, width), jnp.float32)
        zb_t = pltpu.VMEM((RPT, width), jnp.float32)
        acc_t = pltpu.VMEM_SHARED((NP, width), jnp.float32)

    scratch = [pltpu.VMEM((NROW + 1, CHUNK), jnp.int32)]
    if mode != "deg":
        scratch.append(pltpu.VMEM((NROW + 1, CHUNK), jnp.int32))
    if mode == "vreg":
        scratch.append(pltpu.VMEM((N,), jnp.float32))
    scratch += [rows_t, zb_t, acc_t]
    scratch += [pltpu.SemaphoreType.DMA for _ in range(RING)]
    if mode == "stream":
        scratch += [pltpu.SemaphoreType.DMA for _ in range(RING)]

    @functools.partial(
        pl.kernel,
        out_type=out_t,
        mesh=_MESH,
        compiler_params=_SC_PARAMS,
        scratch_types=scratch,
    )
    def agg(*refs):
        it = iter(refs)
        if mode == "deg":
            dst2_hbm, out_hbm = next(it), next(it)
        else:
            table_hbm, src2_hbm, dst2_hbm, out_hbm = (
                next(it), next(it), next(it), next(it))
        idx_d2 = next(it)
        if mode != "deg":
            idx_s2 = next(it)
        if mode == "vreg":
            tab_v = next(it)
        rows_v, zb_v, acc_sh = next(it), next(it), next(it)
        ssem = [next(it) for _ in range(RING)]
        if mode == "stream":
            gsem = [next(it) for _ in range(RING)]

        cid = lax.axis_index("c")
        sid = lax.axis_index("s")
        wid = cid * NS + sid
        has_extra = wid < XROW

        pltpu.sync_copy(dst2_hbm.at[pl.ds(wid * NROW, NROW), :],
                        idx_d2.at[pl.ds(0, NROW), :])
        if mode != "deg":
            pltpu.sync_copy(src2_hbm.at[pl.ds(wid * NROW, NROW), :],
                            idx_s2.at[pl.ds(0, NROW), :])

        @pl.when(has_extra)
        def _():
            xr = NW * NROW + wid
            pltpu.sync_copy(dst2_hbm.at[xr], idx_d2.at[NROW])
            if mode != "deg":
                pltpu.sync_copy(src2_hbm.at[xr], idx_s2.at[NROW])

        if mode == "deg":
            for j in range(CHUNK // 16):
                rows_v[0, pl.ds(j * 16, 16)] = jnp.ones((16,), jnp.float32)
        if mode == "vreg":
            pltpu.sync_copy(table_hbm, tab_v)

        _zero_rows(zb_v, RPT, width)
        pltpu.sync_copy(zb_v, acc_sh.at[pl.ds(sid * RPT, RPT)])
        plsc.subcore_barrier()

        def gather_start(c, slot):
            pltpu.async_copy(table_hbm.at[idx_s2.at[c]], rows_v.at[slot],
                             gsem[slot])

        def gather_wait(c, slot):
            pltpu.make_async_copy(table_hbm.at[idx_s2.at[c]],
                                  rows_v.at[slot], gsem[slot]).wait()

        def vreg_fill(c, slot):
            for k in range(CHUNK // 16):
                sv = idx_s2[c, pl.ds(k * 16, 16)]
                rows_v[slot, pl.ds(k * 16, 16)] = plsc.load_gather(
                    tab_v, [sv])

        def scatter_start(c, slot):
            src = rows_v.at[slot] if mode != "deg" else rows_v.at[0]
            pltpu.async_copy(src, acc_sh.at[idx_d2.at[c]], ssem[slot],
                             add=True)

        def scatter_wait(c, slot):
            src = rows_v.at[slot] if mode != "deg" else rows_v.at[0]
            pltpu.make_async_copy(src, acc_sh.at[idx_d2.at[c]],
                                  ssem[slot]).wait()

        if mode == "stream":
            for j in range(LOOK):
                gather_start(j, j)

            def outer(g, carry):
                for j in range(RING):
                    i = g * RING + j
                    look_slot = (j + LOOK) % RING
                    c = i + LOOK

                    @pl.when(c < NROW)
                    def _():
                        @pl.when(c >= RING)
                        def _():
                            scatter_wait(c, look_slot)
                        gather_start(c, look_slot)

                    gather_wait(i, j)
                    scatter_start(i, j)
                return carry

            lax.fori_loop(0, NROW // RING, outer, 0)
        else:
            def outer(g, carry):
                for j in range(RING):
                    i = g * RING + j

                    @pl.when(i >= RING)
                    def _():
                        scatter_wait(i, j)
                    if mode == "vreg":
                        vreg_fill(i, j)
                    scatter_start(i, j)
                return carry

            lax.fori_loop(0, NROW // RING, outer, 0)

        for j in range(RING):
            scatter_wait(0, j)

        @pl.when(has_extra)
        def _():
            if mode == "stream":
                gather_start(NROW, 0)
                gather_wait(NROW, 0)
            if mode == "vreg":
                vreg_fill(NROW, 0)
            scatter_start(NROW, 0)
            scatter_wait(NROW, 0)

        plsc.subcore_barrier()
        pltpu.sync_copy(
            acc_sh.at[pl.ds(sid * RPT, RPT)],
            out_hbm.at[cid, pl.ds(sid * RPT, RPT)],
        )

    return agg


_deg_kernel = _make_edge_agg("deg")
_agg16 = _make_edge_agg("stream")


@functools.partial(
    pl.kernel,
    out_type=jax.ShapeDtypeStruct((N,), jnp.float32),
    mesh=_MESH,
    compiler_params=_SC_PARAMS,
    scratch_types=[
        pltpu.VMEM((NR2 + 1, CHUNK), jnp.int32),
        pltpu.VMEM((NR2 + 1, CHUNK), jnp.int32),
        pltpu.VMEM((NP,), jnp.float32),
        pltpu.VMEM((OPT,), jnp.float32),
        pltpu.VMEM((16,), jnp.float32),
        pltpu.VMEM((RING, CHUNK), jnp.float32),
        pltpu.VMEM((RPT,), jnp.float32),
        pltpu.VMEM((OPT,), jnp.float32),
        pltpu.VMEM_SHARED((NP,), jnp.float32),
    ] + [pltpu.SemaphoreType.DMA for _ in range(RING)],
)
def _agg1_final(t2_hbm, dis_hbm, b2_hbm, src2_hbm, dst2_hbm, out_hbm,
                idx_d2, idx_s2, tab_v, dis_v, b2_v, rows_v, zb_v, out_v,
                acc_sh, *ssem):
    cid = lax.axis_index("c")
    sid = lax.axis_index("s")
    lo = cid * HALF
    has_extra = sid < XR2

    pltpu.sync_copy(dst2_hbm.at[pl.ds(sid * NR2, NR2), :],
                    idx_d2.at[pl.ds(0, NR2), :])
    pltpu.sync_copy(src2_hbm.at[pl.ds(sid * NR2, NR2), :],
                    idx_s2.at[pl.ds(0, NR2), :])

    @pl.when(has_extra)
    def _():
        xr = NS * NR2 + sid
        pltpu.sync_copy(dst2_hbm.at[xr], idx_d2.at[NR2])
        pltpu.sync_copy(src2_hbm.at[xr], idx_s2.at[NR2])

    pltpu.sync_copy(t2_hbm, tab_v)
    pltpu.sync_copy(dis_hbm.at[pl.ds(lo + sid * OPT, OPT)], dis_v)
    pltpu.sync_copy(b2_hbm, b2_v)

    _zero_rows(zb_v, RPT, 1)
    pltpu.sync_copy(zb_v, acc_sh.at[pl.ds(sid * RPT, RPT)])
    plsc.subcore_barrier()

    def fill(c, slot):
        for k in range(CHUNK // 16):
            sv = idx_s2[c, pl.ds(k * 16, 16)]
            rows_v[slot, pl.ds(k * 16, 16)] = plsc.load_gather(tab_v, [sv])

    def scatter_start(c, slot):
        pltpu.async_copy(rows_v.at[slot], acc_sh.at[idx_d2.at[c]],
                         ssem[slot], add=True)

    def scatter_wait(c, slot):
        pltpu.make_async_copy(rows_v.at[slot], acc_sh.at[idx_d2.at[c]],
                              ssem[slot]).wait()

    def outer(g, carry):
        for j in range(RING):
            i = g * RING + j

            @pl.when(i >= RING)
            def _():
                scatter_wait(i, j)
            fill(i, j)
            scatter_start(i, j)
        return carry

    lax.fori_loop(0, NR2 // RING, outer, 0)
    for j in range(RING):
        scatter_wait(0, j)

    @pl.when(has_extra)
    def _():
        fill(NR2, 0)
        scatter_start(NR2, 0)
        scatter_wait(NR2, 0)

    plsc.subcore_barrier()

    pltpu.sync_copy(acc_sh.at[pl.ds(lo + sid * OPT, OPT)], out_v)
    b2vec = b2_v[...]
    nb = lo + sid * OPT
    for k in range(OPT // 16):
        sl = pl.ds(k * 16, 16)
        t2l = tab_v[pl.ds(nb + k * 16, 16)]
        out_v[sl] = dis_v[sl] * (out_v[sl] + t2l) + b2vec

    base = lo + sid * OPT

    @pl.when(base + OPT <= N)
    def _():
        pltpu.sync_copy(out_v, out_hbm.at[pl.ds(base, OPT)])

    @pl.when(base + OPT > N)
    def _():
        pltpu.sync_copy(out_v.at[pl.ds(0, N - (NC * NS - 1) * OPT)],
                        out_hbm.at[pl.ds(base, N - (NC * NS - 1) * OPT)])


def _tc_stage1(x, w1, d0, d1):
    def body(x_ref, w_ref, d0_ref, d1_ref, dis_ref, g1_ref):
        dis = lax.rsqrt(d0_ref[0:N, :] + d1_ref[0:N, :] + 1.0)
        h = jnp.dot(x_ref[...], w_ref[...], preferred_element_type=jnp.float32)
        dis_ref[0:N, :] = dis
        g1_ref[...] = dis * h

    return pl.pallas_call(
        body,
        out_shape=[
            jax.ShapeDtypeStruct((NP, 1), jnp.float32),
            jax.ShapeDtypeStruct((N, H), jnp.float32),
        ],
    )(x, w1, d0, d1)


def _tc_stage2(acc0, acc1, g1, dis, b1, w2):
    def body(a0_ref, a1_ref, g1_ref, dis_ref, b1_ref, w2_ref, t2_ref):
        agg = a0_ref[0:N, :] + a1_ref[0:N, :] + g1_ref[...]
        disn = dis_ref[0:N, :]
        out1 = disn * agg + b1_ref[...]
        h1 = jnp.maximum(out1, 0.0)
        g2 = jnp.dot(h1, w2_ref[...], preferred_element_type=jnp.float32)
        t2_ref[0:N, :] = disn * g2

    return pl.pallas_call(
        body,
        out_shape=jax.ShapeDtypeStruct((NP, 1), jnp.float32),
    )(acc0, acc1, g1, dis, b1, w2)


def kernel(x, edge_index, W1, b1, W2, b2):
    src2 = edge_index[0].reshape(EROWS, CHUNK)
    dst2 = edge_index[1].reshape(EROWS, CHUNK)

    degp = _deg_kernel(dst2)
    dis, g1 = _tc_stage1(x, W1,
                         degp[0].reshape(NP, 1),
                         degp[1].reshape(NP, 1))
    accp = _agg16(g1, src2, dst2)
    t2 = _tc_stage2(accp[0], accp[1], g1, dis,
                    b1.reshape(1, H), W2)
    out = _agg1_final(t2.reshape(NP), dis.reshape(NP),
                      jnp.tile(b2, 16), src2, dst2)
    return out.reshape(N, 1)


# speedup vs baseline: 1.0091x; 1.0025x over previous
import functools

import jax
import jax.numpy as jnp
from jax import lax
from jax.experimental import pallas as pl
from jax.experimental.pallas import tpu as pltpu
from jax.experimental.pallas import tpu_sc as plsc

N = 10000
NP = 10240
E = 320000
D = 128
H = 16

NC = 2
NS = 16
NW = NC * NS
RPT = NP // NS
CHUNK = 128
EROWS = E // CHUNK
NROW = EROWS // NW
XROW = EROWS - NROW * NW
RING = 13
LOOK = 8

HALF = NP // NC
NR2 = EROWS // NS
XR2 = EROWS - NR2 * NS
OPT = HALF // NS

_MESH = plsc.VectorSubcoreMesh(
    core_axis_name="c", subcore_axis_name="s", num_cores=NC, num_subcores=NS
)
_SC_PARAMS = pltpu.CompilerParams(use_tc_tiling_on_sc=False,
                                  needs_layout_passes=False)


def _zero_rows(ref, nrows, width):
    if width == 1:
        def body(i, carry):
            ref[pl.ds(i * 16, 16)] = jnp.zeros((16,), jnp.float32)
            return carry
        lax.fori_loop(0, nrows // 16, body, 0)
    else:
        def body(i, carry):
            for j in range(width // 16):
                ref[i, pl.ds(j * 16, 16)] = jnp.zeros((16,), jnp.float32)
            return carry
        lax.fori_loop(0, nrows, body, 0)


def _make_edge_agg(mode):
    width = H if mode == "stream" else 1
    if width == 1:
        out_t = jax.ShapeDtypeStruct((NC, NP), jnp.float32)
        rows_t = pltpu.VMEM((RING, CHUNK), jnp.float32)
        zb_t = pltpu.VMEM((RPT,), jnp.float32)
        acc_t = pltpu.VMEM_SHARED((NP,), jnp.float32)
    else:
        out_t = jax.ShapeDtypeStruct((NC, NP, width), jnp.float32)
        rows_t = pltpu.VMEM((RING, CHUNK, width), jnp.float32)
        zb_t = pltpu.VMEM((RPT, width), jnp.float32)
        acc_t = pltpu.VMEM_SHARED((NP, width), jnp.float32)

    scratch = [pltpu.VMEM((NROW + 1, CHUNK), jnp.int32)]
    if mode != "deg":
        scratch.append(pltpu.VMEM((NROW + 1, CHUNK), jnp.int32))
    scratch += [rows_t, zb_t, acc_t]
    scratch += [pltpu.SemaphoreType.DMA for _ in range(RING)]
    if mode == "stream":
        scratch += [pltpu.SemaphoreType.DMA for _ in range(RING)]

    @functools.partial(
        pl.kernel,
        out_type=out_t,
        mesh=_MESH,
        compiler_params=_SC_PARAMS,
        scratch_types=scratch,
    )
    def agg(*refs):
        it = iter(refs)
        if mode == "deg":
            dst2_hbm, out_hbm = next(it), next(it)
        else:
            table_hbm, src2_hbm, dst2_hbm, out_hbm = (
                next(it), next(it), next(it), next(it))
        idx_d2 = next(it)
        if mode != "deg":
            idx_s2 = next(it)
        rows_v, zb_v, acc_sh = next(it), next(it), next(it)
        ssem = [next(it) for _ in range(RING)]
        if mode == "stream":
            gsem = [next(it) for _ in range(RING)]

        cid = lax.axis_index("c")
        sid = lax.axis_index("s")
        wid = cid * NS + sid
        has_extra = wid < XROW

        pltpu.sync_copy(dst2_hbm.at[pl.ds(wid * NROW, NROW), :],
                        idx_d2.at[pl.ds(0, NROW), :])
        if mode != "deg":
            pltpu.sync_copy(src2_hbm.at[pl.ds(wid * NROW, NROW), :],
                            idx_s2.at[pl.ds(0, NROW), :])

        @pl.when(has_extra)
        def _():
            xr = NW * NROW + wid
            pltpu.sync_copy(dst2_hbm.at[xr], idx_d2.at[NROW])
            if mode != "deg":
                pltpu.sync_copy(src2_hbm.at[xr], idx_s2.at[NROW])

        if mode == "deg":
            for j in range(CHUNK // 16):
                rows_v[0, pl.ds(j * 16, 16)] = jnp.ones((16,), jnp.float32)

        _zero_rows(zb_v, RPT, width)
        pltpu.sync_copy(zb_v, acc_sh.at[pl.ds(sid * RPT, RPT)])
        plsc.subcore_barrier()

        def gather_start(c, slot):
            pltpu.async_copy(table_hbm.at[idx_s2.at[c]], rows_v.at[slot],
                             gsem[slot])

        def gather_wait(c, slot):
            pltpu.make_async_copy(table_hbm.at[idx_s2.at[c]],
                                  rows_v.at[slot], gsem[slot]).wait()

        def scatter_start(c, slot):
            src = rows_v.at[slot] if mode != "deg" else rows_v.at[0]
            pltpu.async_copy(src, acc_sh.at[idx_d2.at[c]], ssem[slot],
                             add=True)

        def scatter_wait(c, slot):
            src = rows_v.at[slot] if mode != "deg" else rows_v.at[0]
            pltpu.make_async_copy(src, acc_sh.at[idx_d2.at[c]],
                                  ssem[slot]).wait()

        if mode == "stream":
            for j in range(LOOK):
                gather_start(j, j)

            def outer(g, carry):
                for j in range(RING):
                    i = g * RING + j
                    look_slot = (j + LOOK) % RING
                    c = i + LOOK

                    @pl.when(c < NROW)
                    def _():
                        @pl.when(c >= RING)
                        def _():
                            scatter_wait(c, look_slot)
                        gather_start(c, look_slot)

                    gather_wait(i, j)
                    scatter_start(i, j)
                return carry

            lax.fori_loop(0, NROW // RING, outer, 0)
        else:
            def outer(g, carry):
                for j in range(RING):
                    i = g * RING + j

                    @pl.when(i >= RING)
                    def _():
                        scatter_wait(i, j)
                    scatter_start(i, j)
                return carry

            lax.fori_loop(0, NROW // RING, outer, 0)

        for j in range(RING):
            scatter_wait(0, j)

        @pl.when(has_extra)
        def _():
            if mode == "stream":
                gather_start(NROW, 0)
                gather_wait(NROW, 0)
            scatter_start(NROW, 0)
            scatter_wait(NROW, 0)

        plsc.subcore_barrier()
        pltpu.sync_copy(
            acc_sh.at[pl.ds(sid * RPT, RPT)],
            out_hbm.at[cid, pl.ds(sid * RPT, RPT)],
        )

    return agg


_deg_kernel = _make_edge_agg("deg")
_agg16 = _make_edge_agg("stream")


@functools.partial(
    pl.kernel,
    out_type=jax.ShapeDtypeStruct((N,), jnp.float32),
    mesh=_MESH,
    compiler_params=_SC_PARAMS,
    scratch_types=[
        pltpu.VMEM((NR2 + 1, CHUNK), jnp.int32),
        pltpu.VMEM((NR2 + 1, CHUNK), jnp.int32),
        pltpu.VMEM((NP,), jnp.float32),
        pltpu.VMEM((OPT,), jnp.float32),
        pltpu.VMEM((16,), jnp.float32),
        pltpu.VMEM((RING, CHUNK), jnp.float32),
        pltpu.VMEM((RPT,), jnp.float32),
        pltpu.VMEM((OPT,), jnp.float32),
        pltpu.VMEM_SHARED((NP,), jnp.float32),
    ] + [pltpu.SemaphoreType.DMA for _ in range(RING)],
)
def _agg1_final(t2_hbm, dis_hbm, b2_hbm, src2_hbm, dst2_hbm, out_hbm,
                idx_d2, idx_s2, tab_v, dis_v, b2_v, rows_v, zb_v, out_v,
                acc_sh, *ssem):
    cid = lax.axis_index("c")
    sid = lax.axis_index("s")
    lo = cid * HALF
    has_extra = sid < XR2

    pltpu.sync_copy(dst2_hbm.at[pl.ds(sid * NR2, NR2), :],
                    idx_d2.at[pl.ds(0, NR2), :])
    pltpu.sync_copy(src2_hbm.at[pl.ds(sid * NR2, NR2), :],
                    idx_s2.at[pl.ds(0, NR2), :])

    @pl.when(has_extra)
    def _():
        xr = NS * NR2 + sid
        pltpu.sync_copy(dst2_hbm.at[xr], idx_d2.at[NR2])
        pltpu.sync_copy(src2_hbm.at[xr], idx_s2.at[NR2])

    pltpu.sync_copy(t2_hbm, tab_v)
    pltpu.sync_copy(dis_hbm.at[pl.ds(lo + sid * OPT, OPT)], dis_v)
    pltpu.sync_copy(b2_hbm, b2_v)

    _zero_rows(zb_v, RPT, 1)
    pltpu.sync_copy(zb_v, acc_sh.at[pl.ds(sid * RPT, RPT)])
    plsc.subcore_barrier()

    def fill(c, slot):
        for k in range(CHUNK // 16):
            sv = idx_s2[c, pl.ds(k * 16, 16)]
            rows_v[slot, pl.ds(k * 16, 16)] = plsc.load_gather(tab_v, [sv])

    def scatter_start(c, slot):
        pltpu.async_copy(rows_v.at[slot], acc_sh.at[idx_d2.at[c]],
                         ssem[slot], add=True)

    def scatter_wait(c, slot):
        pltpu.make_async_copy(rows_v.at[slot], acc_sh.at[idx_d2.at[c]],
                              ssem[slot]).wait()

    def outer(g, carry):
        for j in range(RING):
            i = g * RING + j

            @pl.when(i >= RING)
            def _():
                scatter_wait(i, j)
            fill(i, j)
            scatter_start(i, j)
        return carry

    lax.fori_loop(0, NR2 // RING, outer, 0)
    for j in range(RING):
        scatter_wait(0, j)

    @pl.when(has_extra)
    def _():
        fill(NR2, 0)
        scatter_start(NR2, 0)
        scatter_wait(NR2, 0)

    plsc.subcore_barrier()

    pltpu.sync_copy(acc_sh.at[pl.ds(lo + sid * OPT, OPT)], out_v)
    b2vec = b2_v[...]
    nb = lo + sid * OPT
    for k in range(OPT // 16):
        sl = pl.ds(k * 16, 16)
        t2l = tab_v[pl.ds(nb + k * 16, 16)]
        out_v[sl] = dis_v[sl] * (out_v[sl] + t2l) + b2vec

    base = lo + sid * OPT

    @pl.when(base + OPT <= N)
    def _():
        pltpu.sync_copy(out_v, out_hbm.at[pl.ds(base, OPT)])

    @pl.when(base + OPT > N)
    def _():
        pltpu.sync_copy(out_v.at[pl.ds(0, N - (NC * NS - 1) * OPT)],
                        out_hbm.at[pl.ds(base, N - (NC * NS - 1) * OPT)])


def _tc_stage1(x, w1, d0, d1):
    def body(x_ref, w_ref, d0_ref, d1_ref, dis_ref, g1_ref):
        dis = lax.rsqrt(d0_ref[0:N, :] + d1_ref[0:N, :] + 1.0)
        h = jnp.dot(x_ref[...], w_ref[...], preferred_element_type=jnp.float32)
        dis_ref[0:N, :] = dis
        g1_ref[...] = dis * h

    return pl.pallas_call(
        body,
        out_shape=[
            jax.ShapeDtypeStruct((NP, 1), jnp.float32),
            jax.ShapeDtypeStruct((N, H), jnp.float32),
        ],
    )(x, w1, d0, d1)


def _tc_stage2(acc0, acc1, g1, dis, b1, w2):
    def body(a0_ref, a1_ref, g1_ref, dis_ref, b1_ref, w2_ref, t2_ref):
        agg = a0_ref[0:N, :] + a1_ref[0:N, :] + g1_ref[...]
        disn = dis_ref[0:N, :]
        out1 = disn * agg + b1_ref[...]
        h1 = jnp.maximum(out1, 0.0)
        g2 = jnp.dot(h1, w2_ref[...], preferred_element_type=jnp.float32)
        t2_ref[0:N, :] = disn * g2

    return pl.pallas_call(
        body,
        out_shape=jax.ShapeDtypeStruct((NP, 1), jnp.float32),
    )(acc0, acc1, g1, dis, b1, w2)


def kernel(x, edge_index, W1, b1, W2, b2):
    src2 = edge_index[0].reshape(EROWS, CHUNK)
    dst2 = edge_index[1].reshape(EROWS, CHUNK)

    degp = _deg_kernel(dst2)
    dis, g1 = _tc_stage1(x, W1,
                         degp[0].reshape(NP, 1),
                         degp[1].reshape(NP, 1))
    accp = _agg16(g1, src2, dst2)
    t2 = _tc_stage2(accp[0], accp[1], g1, dis,
                    b1.reshape(1, H), W2)
    out = _agg1_final(t2.reshape(NP), dis.reshape(NP),
                      jnp.tile(b2, 16), src2, dst2)
    return out.reshape(N, 1)
